# Initial kernel scaffold; baseline (speedup 1.0000x reference)
import jax
import jax.numpy as jnp
from jax.experimental import pallas as pl


def kernel(point_coords, feats, bboxes):
    raise NotImplementedError("write your pallas kernel here")


# speedup vs baseline: 4.4782x; 4.4782x over previous
import functools

import jax
import jax.numpy as jnp
from jax.experimental import pallas as pl

_REFINE = 3
_TAU = 0.85
_N = 128
_P = 10
_C = 384
_H = 32
_W = 32
_HW = _H * _W
_NB = 96
_CHUNK = 8

_dot = functools.partial(
    jax.lax.dot_general,
    precision=jax.lax.Precision.HIGHEST,
    preferred_element_type=jnp.float32,
)


def _cos_num(ff, vecs):
    rows = []
    for i in range(_N // _CHUNK):
        v = vecs[i * _CHUNK : (i + 1) * _CHUNK, :]
        prod = ff[None, :, :] * v[:, None, :]
        rows.append(jnp.sum(prod, axis=2))
    return jnp.concatenate(rows, axis=0)


def _dpm_kernel(px_ref, py_ref, bb_ref, ff_ref, fft_ref, out_ref):
    ff = ff_ref[...]
    fft = fft_ref[...]

    fnorm0 = jnp.sqrt(jnp.sum(ff * ff, axis=1, keepdims=True)).T
    fnorm_r = jnp.sqrt(jnp.sum(fft * fft, axis=0, keepdims=True))

    ix = jnp.clip(jnp.floor(px_ref[...] / 16.0), 0.0, _W - 1.0)
    iy = jnp.clip(jnp.floor(py_ref[...] / 16.0), 0.0, _H - 1.0)
    hw = (iy * _W + ix).astype(jnp.int32)
    iota = jax.lax.broadcasted_iota(jnp.int32, (_N, _HW), 1)
    pf_sum = jnp.zeros((_N, _C), jnp.float32)
    for p in range(_P):
        onehot = jnp.where(hw[:, p : p + 1] == iota, 1.0, 0.0)
        pf_sum = pf_sum + _dot(onehot, ff, (((1,), (0,)), ((), ())))
    pf_mean = pf_sum / float(_P)
    pf_norm = jnp.sqrt(jnp.sum(pf_mean * pf_mean, axis=1, keepdims=True))

    num0 = _cos_num(ff, pf_mean)
    cm = num0 / jnp.maximum(fnorm0 * pf_norm, 1e-8)

    b = jnp.floor(bb_ref[...] / 16.0)
    r = (iota // _W).astype(jnp.float32)
    c = (iota % _W).astype(jnp.float32)
    mask = (
        (r >= b[:, 1:2]) & (r <= b[:, 3:4]) & (c >= b[:, 0:1]) & (c <= b[:, 2:3])
    ).astype(jnp.float32)

    nidx = jax.lax.broadcasted_iota(jnp.int32, (_N, _HW), 0)

    def select(cmi):
        cmm = cmi * mask
        colmax = jnp.max(cmm, axis=0, keepdims=True)
        elig = cmm == colmax
        first = jnp.min(jnp.where(elig, nidx, _N), axis=0, keepdims=True)
        sel = jnp.where(nidx == first, cmm, 0.0)
        return cmm, sel

    _, sel0 = select(cm)
    out_ref[0] = sel0

    cm1 = cm
    for t in range(_REFINE):
        mx = jnp.max(cm1, axis=1, keepdims=True)
        thr = mx * _TAU
        cm1 = jnp.where(cm1 < thr, 0.0, cm1)
        nz = (cm1 != 0.0).astype(jnp.float32)
        nsurv = jnp.sum(nz, axis=1, keepdims=True)
        rs = jnp.sum(cm1, axis=1, keepdims=True)
        den_row = jnp.maximum(rs, 1e-8)
        fm_gen = _dot(cm1, ff, (((1,), (0,)), ((), ()))) / den_row
        onemax = jnp.where((cm1 == mx) & (nz != 0.0), 1.0, 0.0)
        g = _dot(onemax, ff, (((1,), (0,)), ((), ())))
        fm_one = (mx * g) / den_row
        fm = jnp.where(nsurv == 1.0, fm_one, fm_gen)
        fmt = fm.T
        fmn = jnp.sqrt(jnp.sum(fmt * fmt, axis=0, keepdims=True)).T
        numt = _cos_num(ff, fm)
        cmt = numt / jnp.maximum(fnorm_r * fmn, 1e-8)
        cm1, sel = select(cmt)
        out_ref[t + 1] = sel


@jax.jit
def kernel(point_coords, feats, bboxes):
    px = point_coords[..., 0]
    py = point_coords[..., 1]
    fft = feats[0].reshape(_C, _HW)
    ff = jnp.transpose(feats[0], (1, 2, 0)).reshape(_HW, _C)
    pad = jnp.tile(jnp.array([[0.0, 0.0, 1e9, 1e9]], jnp.float32), (_N - _NB, 1))
    bb = jnp.concatenate([bboxes, pad], axis=0)
    out = pl.pallas_call(
        _dpm_kernel,
        out_shape=jax.ShapeDtypeStruct((_REFINE + 1, _N, _HW), jnp.float32),
    )(px, py, bb, ff, fft)
    return out.reshape(_REFINE + 1, _N, _H, _W)
